# baseline (speedup 1.0000x reference)
import functools

import jax
import jax.numpy as jnp
from jax import lax
from jax.experimental import pallas as pl
from jax.experimental.pallas import tpu as pltpu
from jax.experimental.pallas import tpu_sc as plsc

N = 10000
D = 128
G = 128
NC = 2
NS = 16
L = 16
BLK = 1000
CH = 80
ROWT = 624

@functools.cache
def _get_mesh():
    return plsc.VectorSubcoreMesh(core_axis_name="c", subcore_axis_name="s",
                                  num_cores=NC, num_subcores=NS)


def _worker(cid, sid):
    return cid * NS + sid


def _zero_acc(zeros_hbm, acc, sid):
    pltpu.sync_copy(zeros_hbm, acc.at[pl.ds(sid * ROWT, ROWT)])

    @pl.when(sid == NS - 1)
    def _():
        pltpu.sync_copy(zeros_hbm.at[pl.ds(0, N - NS * ROWT)],
                        acc.at[pl.ds(NS * ROWT, N - NS * ROWT)])


def _copy_out(acc, out_hbm, cid, sid):
    pltpu.sync_copy(acc.at[pl.ds(sid * ROWT, ROWT)],
                    out_hbm.at[cid, pl.ds(sid * ROWT, ROWT)])

    @pl.when(sid == NS - 1)
    def _():
        pltpu.sync_copy(acc.at[pl.ds(NS * ROWT, N - NS * ROWT)],
                        out_hbm.at[cid, pl.ds(NS * ROWT, N - NS * ROWT)])


def _deg_body(dst_hbm, zeros_hbm, out_hbm, dst0, dst1, onesv, acc,
              semd0, semd1):
    cid = lax.axis_index("c")
    sid = lax.axis_index("s")
    e_total = dst_hbm.shape[0]
    e_core = e_total // NC
    e_tile = e_core // NS
    nchunks = e_tile // CH

    ones = jnp.ones((L,), jnp.float32)
    for r in range(CH):
        onesv[r, :] = ones
    _zero_acc(zeros_hbm, acc, sid)
    plsc.subcore_barrier()

    tile_base = cid * e_core + sid * e_tile
    dstb = (dst0, dst1)
    semdb = (semd0, semd1)

    def lddst(j, b):
        pltpu.async_copy(dst_hbm.at[pl.ds(tile_base + j * CH, CH)],
                         dstb[b], semdb[b])

    def scat(j, b):
        pltpu.make_async_copy(dst_hbm.at[pl.ds(tile_base + j * CH, CH)],
                              dstb[b], semdb[b]).wait()
        pltpu.sync_copy(onesv, acc.at[dstb[b]], add=True)

    lddst(0, 0)
    lddst(1, 1)

    def step(s, carry):
        j = 2 * s
        scat(j, 0)

        @pl.when(j + 2 < nchunks)
        def _():
            lddst(j + 2, 0)

        scat(j + 1, 1)

        @pl.when(j + 3 < nchunks)
        def _():
            lddst(j + 3, 1)

        return carry

    lax.fori_loop(0, (nchunks - 1) // 2, step, 0)
    scat(nchunks - 1, 0)
    plsc.subcore_barrier()
    _copy_out(acc, out_hbm, cid, sid)


def _sc_degree(dst):
    zeros = jnp.zeros((ROWT, L), jnp.float32)
    return pl.kernel(
        _deg_body,
        out_type=jax.ShapeDtypeStruct((NC, N, L), jnp.float32),
        mesh=_get_mesh(),
        scratch_types=[
            pltpu.VMEM((CH,), jnp.int32),
            pltpu.VMEM((CH,), jnp.int32),
            pltpu.VMEM((CH, L), jnp.float32),
            pltpu.VMEM_SHARED((N, L), jnp.float32),
            pltpu.SemaphoreType.DMA,
            pltpu.SemaphoreType.DMA,
        ],
        compiler_params=pltpu.CompilerParams(use_tc_tiling_on_sc=False),
    )(dst, zeros)


def _agg_body(table_hbm, src_hbm, dst_hbm, zeros_hbm, out_hbm,
              src0, src1, dst0, dst1, dst2, dst3, rows0, rows1, acc,
              semg0, semg1, sems0, sems1,
              semd0, semd1, semd2, semd3, semsc0, semsc1):
    cid = lax.axis_index("c")
    sid = lax.axis_index("s")
    e_total = src_hbm.shape[0]
    e_core = e_total // NC
    e_tile = e_core // NS
    nchunks = e_tile // CH

    _zero_acc(zeros_hbm, acc, sid)
    tile_base = cid * e_core + sid * e_tile
    plsc.subcore_barrier()

    srcb = (src0, src1)
    dstb = (dst0, dst1, dst2, dst3)
    rowsb = (rows0, rows1)
    semgb = (semg0, semg1)
    semsb = (sems0, sems1)
    semdb = (semd0, semd1, semd2, semd3)
    semscb = (semsc0, semsc1)

    def ldsrc(j, b):
        pltpu.async_copy(src_hbm.at[pl.ds(tile_base + j * CH, CH)],
                         srcb[b], semsb[b])

    def lddst(j, b):
        pltpu.async_copy(dst_hbm.at[pl.ds(tile_base + j * CH, CH)],
                         dstb[b], semdb[b])

    def wait_scatter(rb):
        pltpu.make_async_copy(rowsb[rb], acc.at[dstb[0]], semscb[rb]).wait()

    def gather(j, rb, wait_sc):
        pltpu.make_async_copy(src_hbm.at[pl.ds(tile_base + j * CH, CH)],
                              srcb[rb], semsb[rb]).wait()
        if wait_sc is True:
            wait_scatter(rb)
        elif wait_sc is not None:
            @pl.when(wait_sc)
            def _():
                wait_scatter(rb)
        pltpu.async_copy(table_hbm.at[srcb[rb]], rowsb[rb], semgb[rb])

    def scat(j, rb, db):
        pltpu.make_async_copy(table_hbm.at[srcb[rb]],
                              rowsb[rb], semgb[rb]).wait()
        pltpu.make_async_copy(dst_hbm.at[pl.ds(tile_base + j * CH, CH)],
                              dstb[db], semdb[db]).wait()
        pltpu.async_copy(rowsb[rb], acc.at[dstb[db]], semscb[rb], add=True)

    ldsrc(0, 0)
    ldsrc(1, 1)
    lddst(0, 0)
    lddst(1, 1)
    lddst(2, 2)
    gather(0, 0, wait_sc=None)

    def step(s, carry):
        j = 4 * s
        for p in range(4):
            c = j + p
            gather(c + 1, (p + 1) % 2, wait_sc=(j > 0) if p == 0 else True)
            scat(c, p % 2, p % 4)

            @pl.when(c + 2 < nchunks)
            def _(c=c, p=p):
                ldsrc(c + 2, p % 2)

            @pl.when(c + 3 < nchunks)
            def _(c=c, p=p):
                lddst(c + 3, (p + 3) % 4)

        return carry

    lax.fori_loop(0, (nchunks - 1) // 4, step, 0)
    scat(nchunks - 1, (nchunks - 1) % 2, (nchunks - 1) % 4)
    wait_scatter(0)
    wait_scatter(1)
    plsc.subcore_barrier()
    _copy_out(acc, out_hbm, cid, sid)


def _gine_body(table_hbm, src_hbm, dst_hbm, e_hbm, zeros_hbm, out_hbm,
               src0, src1, dst0, dst1, rows0, rows1, acc,
               semg0, semg1, seme0, seme1, sems0, sems1, semd0, semd1):
    cid = lax.axis_index("c")
    sid = lax.axis_index("s")
    e_total = src_hbm.shape[0]
    e_core = e_total // NC
    e_tile = e_core // NS
    nchunks = e_tile // CH

    _zero_acc(zeros_hbm, acc, sid)
    tile_base = cid * e_core + sid * e_tile
    plsc.subcore_barrier()

    srcb = (src0, src1)
    dstb = (dst0, dst1)
    rowsb = (rows0, rows1)
    semgb = (semg0, semg1)
    semeb = (seme0, seme1)
    semsb = (sems0, sems1)
    semdb = (semd0, semd1)

    def ldidx(j, b):
        pltpu.async_copy(src_hbm.at[pl.ds(tile_base + j * CH, CH)],
                         srcb[b], semsb[b])
        pltpu.async_copy(dst_hbm.at[pl.ds(tile_base + j * CH, CH)],
                         dstb[b], semdb[b])

    def e_load(j, b):
        pltpu.async_copy(e_hbm.at[pl.ds(tile_base + j * CH, CH)],
                         rowsb[b], semeb[b])

    def ga(j, b):
        pltpu.make_async_copy(e_hbm.at[pl.ds(tile_base + j * CH, CH)],
                              rowsb[b], semeb[b]).wait()
        pltpu.make_async_copy(src_hbm.at[pl.ds(tile_base + j * CH, CH)],
                              srcb[b], semsb[b]).wait()
        pltpu.async_copy(table_hbm.at[srcb[b]], rowsb[b], semgb[b], add=True)

    def fin(j, b):
        pltpu.make_async_copy(table_hbm.at[srcb[b]],
                              rowsb[b], semgb[b]).wait()
        pltpu.make_async_copy(dst_hbm.at[pl.ds(tile_base + j * CH, CH)],
                              dstb[b], semdb[b]).wait()

        def row_relu(r, c2):
            for k in range(D // L):
                sl = pl.ds(k * L, L)
                rowsb[b][r, sl] = jnp.maximum(rowsb[b][r, sl], 0.0)
            return c2

        lax.fori_loop(0, CH, row_relu, 0, unroll=2)
        pltpu.sync_copy(rowsb[b], acc.at[dstb[b]], add=True)

    ldidx(0, 0)
    ldidx(1, 1)
    e_load(0, 0)
    e_load(1, 1)
    ga(0, 0)

    def step(s, carry):
        j = 2 * s
        ga(j + 1, 1)
        fin(j, 0)

        @pl.when(j + 2 < nchunks)
        def _():
            ldidx(j + 2, 0)
            e_load(j + 2, 0)

        @pl.when(j + 2 < nchunks)
        def _():
            ga(j + 2, 0)

        fin(j + 1, 1)

        @pl.when(j + 3 < nchunks)
        def _():
            ldidx(j + 3, 1)
            e_load(j + 3, 1)

        return carry

    lax.fori_loop(0, (nchunks - 1) // 2, step, 0)
    fin(nchunks - 1, 0)
    plsc.subcore_barrier()
    _copy_out(acc, out_hbm, cid, sid)


def _sc_aggregate(table, src, dst, e_rows):
    with_e = e_rows is not None
    zeros = jnp.zeros((ROWT, D), jnp.float32)
    e_tile = src.shape[0] // (NC * NS)
    nchunks = e_tile // CH
    assert nchunks % 2 == 1 and e_tile % CH == 0
    idx = lambda: pltpu.VMEM((CH,), jnp.int32)
    rows = lambda: pltpu.VMEM((CH, D), jnp.float32)
    sem = pltpu.SemaphoreType.DMA
    if with_e:
        scratch = ([idx(), idx(), idx(), idx(), rows(), rows(),
                    pltpu.VMEM_SHARED((N, D), jnp.float32)] + [sem] * 8)
        return pl.kernel(
            _gine_body,
            out_type=jax.ShapeDtypeStruct((NC, N, D), jnp.float32),
            mesh=_get_mesh(),
            scratch_types=scratch,
        )(table, src, dst, e_rows, zeros)
    assert (nchunks - 1) % 4 == 0
    scratch = ([idx(), idx(), idx(), idx(), idx(), idx(), rows(), rows(),
                pltpu.VMEM_SHARED((N, D), jnp.float32)] + [sem] * 10)
    return pl.kernel(
        _agg_body,
        out_type=jax.ShapeDtypeStruct((NC, N, D), jnp.float32),
        mesh=_get_mesh(),
        scratch_types=scratch,
    )(table, src, dst, zeros)


def _row_spec(cols=D):
    return pl.BlockSpec((BLK, cols), lambda i: (i, 0))


def _full_spec(shape):
    return pl.BlockSpec(shape, lambda i: tuple(0 for _ in shape))


def _dinv_of(d0_ref, d1_ref):
    deg = d0_ref[:, 0:1] + d1_ref[:, 0:1] + 1.0
    return lax.rsqrt(jnp.maximum(deg, 1.0))


def _embed_body(ea_ref, w_ref, b_ref, out_ref):
    out_ref[...] = jnp.dot(ea_ref[...], w_ref[...],
                           preferred_element_type=jnp.float32) + b_ref[...]


def _tc_embed(edge_attr, w, b):
    e, de = edge_attr.shape
    eblk = 4000
    return pl.pallas_call(
        _embed_body,
        grid=(e // eblk,),
        in_specs=[pl.BlockSpec((eblk, de), lambda i: (i, 0)),
                  _full_spec((de, D)),
                  _full_spec((1, D))],
        out_specs=pl.BlockSpec((eblk, D), lambda i: (i, 0)),
        out_shape=jax.ShapeDtypeStruct((e, D), jnp.float32),
    )(edge_attr, w, b.reshape(1, D))


def _hprime_body(x_ref, w_ref, d0_ref, d1_ref, out_ref):
    out_ref[...] = jnp.dot(x_ref[...], w_ref[...],
                           preferred_element_type=jnp.float32) * _dinv_of(d0_ref, d1_ref)


def _tc_hprime(x, w, d0, d1):
    return pl.pallas_call(
        _hprime_body,
        grid=(N // BLK,),
        in_specs=[_row_spec(), _full_spec((D, D)), _row_spec(L), _row_spec(L)],
        out_specs=_row_spec(),
        out_shape=jax.ShapeDtypeStruct((N, D), jnp.float32),
    )(x, w, d0, d1)


def _b1_body(hp_ref, a0_ref, a1_ref, d0_ref, d1_ref, bg_ref,
             x1_ref, g0_ref, g1_ref, w1_ref, b1_ref, w2_ref, b2_ref,
             gcn_ref, gine_ref, st_ref):
    i = pl.program_id(0)
    gcn = (_dinv_of(d0_ref, d1_ref) * (a0_ref[...] + a1_ref[...] + hp_ref[...])
           + bg_ref[...])
    h = x1_ref[...] + g0_ref[...] + g1_ref[...]
    h = jnp.maximum(jnp.dot(h, w1_ref[...],
                            preferred_element_type=jnp.float32) + b1_ref[...], 0.0)
    gine = jnp.dot(h, w2_ref[...],
                   preferred_element_type=jnp.float32) + b2_ref[...]
    gcn_ref[...] = gcn
    gine_ref[...] = gine
    z = jnp.zeros((1, D), jnp.float32)
    st = jnp.concatenate([
        jnp.sum(gcn, axis=0, keepdims=True),
        jnp.sum(gcn * gcn, axis=0, keepdims=True),
        jnp.sum(gine, axis=0, keepdims=True),
        jnp.sum(gine * gine, axis=0, keepdims=True),
        z, z, z, z], axis=0)

    @pl.when(i == 0)
    def _():
        st_ref[...] = jnp.zeros_like(st_ref)

    st_ref[...] += st


def _tc_b1(hprime, a0, a1, d0, d1, bg, x1, g0, g1, w1, b1, w2, b2):
    return pl.pallas_call(
        _b1_body,
        grid=(N // BLK,),
        in_specs=[_row_spec(), _row_spec(), _row_spec(),
                  _row_spec(L), _row_spec(L),
                  _full_spec((1, D)), _row_spec(), _row_spec(), _row_spec(),
                  _full_spec((D, D)), _full_spec((1, D)),
                  _full_spec((D, D)), _full_spec((1, D))],
        out_specs=[_row_spec(), _row_spec(), _full_spec((8, D))],
        out_shape=[jax.ShapeDtypeStruct((N, D), jnp.float32),
                   jax.ShapeDtypeStruct((N, D), jnp.float32),
                   jax.ShapeDtypeStruct((8, D), jnp.float32)],
    )(hprime, a0, a1, d0, d1, bg.reshape(1, D), x1, g0, g1,
      w1, b1.reshape(1, D), w2, b2.reshape(1, D))


def _pair_attention(x):
    half = x.shape[0] // 2
    xr = x.reshape(half, 2, D)
    a = xr[:, 0, :]
    b = xr[:, 1, :]
    saa = jnp.sum(a * a, axis=-1, keepdims=True)
    sab = jnp.sum(a * b, axis=-1, keepdims=True)
    sbb = jnp.sum(b * b, axis=-1, keepdims=True)
    ma = jnp.maximum(saa, sab)
    ea0 = jnp.exp(saa - ma)
    ea1 = jnp.exp(sab - ma)
    a2 = (ea0 * a + ea1 * b) / (ea0 + ea1)
    mb = jnp.maximum(sab, sbb)
    eb0 = jnp.exp(sab - mb)
    eb1 = jnp.exp(sbb - mb)
    b2 = (eb0 * a + eb1 * b) / (eb0 + eb1)
    return jnp.stack([a2, b2], axis=1).reshape(x.shape[0], D)


def _bn_attn(gcn_ref, gine_ref, st_ref, x0_ref, x1_ref,
             bg0_ref, bb0_ref, bg1_ref, bb1_ref):
    st = st_ref[...]
    n = jnp.float32(N)
    m0 = st[0:1] / n
    v0 = st[1:2] / n - m0 * m0
    m1 = st[2:3] / n
    v1 = st[3:4] / n - m1 * m1
    h0 = (gcn_ref[...] - m0) * lax.rsqrt(v0 + 1e-5) * bg0_ref[...] + bb0_ref[...]
    x0n = x0_ref[...] + jnp.maximum(h0, 0.0)
    h1 = (gine_ref[...] - m1) * lax.rsqrt(v1 + 1e-5) * bg1_ref[...] + bb1_ref[...]
    x1n = x1_ref[...] + jnp.maximum(h1, 0.0)
    return _pair_attention(x0n), _pair_attention(x1n)


def _b2h_body(gcn_ref, gine_ref, st_ref, x0_ref, x1_ref,
              bg0_ref, bb0_ref, bg1_ref, bb1_ref, w_ref, d0_ref, d1_ref,
              o0_ref, o1_ref, hp_ref):
    o0, o1 = _bn_attn(gcn_ref, gine_ref, st_ref, x0_ref, x1_ref,
                      bg0_ref, bb0_ref, bg1_ref, bb1_ref)
    o0_ref[...] = o0
    o1_ref[...] = o1
    hp_ref[...] = jnp.dot(o0, w_ref[...],
                          preferred_element_type=jnp.float32) * _dinv_of(d0_ref, d1_ref)


def _tc_b2h(gcn_pre, gine_pre, st, x0, x1, bg0, bb0, bg1, bb1, w, d0, d1):
    return pl.pallas_call(
        _b2h_body,
        grid=(N // BLK,),
        in_specs=[_row_spec(), _row_spec(), _full_spec((8, D)),
                  _row_spec(), _row_spec(),
                  _full_spec((1, D)), _full_spec((1, D)),
                  _full_spec((1, D)), _full_spec((1, D)),
                  _full_spec((D, D)), _row_spec(L), _row_spec(L)],
        out_specs=[_row_spec(), _row_spec(), _row_spec()],
        out_shape=[jax.ShapeDtypeStruct((N, D), jnp.float32),
                   jax.ShapeDtypeStruct((N, D), jnp.float32),
                   jax.ShapeDtypeStruct((N, D), jnp.float32)],
    )(gcn_pre, gine_pre, st, x0, x1,
      bg0.reshape(1, D), bb0.reshape(1, D), bg1.reshape(1, D), bb1.reshape(1, D),
      w, d0, d1)


def _b2pool_body(gcn_ref, gine_ref, st_ref, x0_ref, x1_ref,
                 bg0_ref, bb0_ref, bg1_ref, bb1_ref, b_ref, ow_ref, ob_ref,
                 out_ref, sums_ref, cnt_ref):
    i = pl.program_id(0)
    nb = pl.num_programs(0)
    o0, o1 = _bn_attn(gcn_ref, gine_ref, st_ref, x0_ref, x1_ref,
                      bg0_ref, bb0_ref, bg1_ref, bb1_ref)
    x = o0 + o1
    seg = b_ref[...]
    iota = lax.broadcasted_iota(jnp.int32, (1, G), 1)
    onehot = (seg == iota).astype(jnp.float32)
    psum = lax.dot_general(onehot, x, (((0,), (0,)), ((), ())),
                           preferred_element_type=jnp.float32)
    ones = jnp.ones((x.shape[0], 1), jnp.float32)
    pcnt = lax.dot_general(onehot, ones, (((0,), (0,)), ((), ())),
                           preferred_element_type=jnp.float32)

    @pl.when(i == 0)
    def _():
        sums_ref[...] = jnp.zeros_like(sums_ref)
        cnt_ref[...] = jnp.zeros_like(cnt_ref)

    sums_ref[...] += psum
    cnt_ref[...] += pcnt

    @pl.when(i == nb - 1)
    def _():
        pooled = sums_ref[...] / jnp.maximum(cnt_ref[...], 1.0)
        out_ref[...] = jnp.dot(pooled, ow_ref[...],
                               preferred_element_type=jnp.float32) + ob_ref[...]


def _tc_b2pool(gcn_pre, gine_pre, st, x0, x1, bg0, bb0, bg1, bb1,
               batch_col, out_w, out_b):
    out_dim = out_w.shape[1]
    out, _, _ = pl.pallas_call(
        _b2pool_body,
        grid=(N // BLK,),
        in_specs=[_row_spec(), _row_spec(), _full_spec((8, D)),
                  _row_spec(), _row_spec(),
                  _full_spec((1, D)), _full_spec((1, D)),
                  _full_spec((1, D)), _full_spec((1, D)),
                  _row_spec(1),
                  _full_spec((D, out_dim)), _full_spec((1, out_dim))],
        out_specs=[_full_spec((G, out_dim)), _full_spec((G, D)),
                   _full_spec((G, 1))],
        out_shape=[jax.ShapeDtypeStruct((G, out_dim), jnp.float32),
                   jax.ShapeDtypeStruct((G, D), jnp.float32),
                   jax.ShapeDtypeStruct((G, 1), jnp.float32)],
    )(gcn_pre, gine_pre, st, x0, x1,
      bg0.reshape(1, D), bb0.reshape(1, D), bg1.reshape(1, D), bb1.reshape(1, D),
      batch_col, out_w, out_b.reshape(1, out_dim))
    return out


def kernel(x0, x1, edge_attr, params, edge_index, batch):
    src = edge_index[0]
    dst = edge_index[1]

    deg_partials = _sc_degree(dst)
    d0, d1 = deg_partials[0], deg_partials[1]

    e_rows = _tc_embed(edge_attr, params['gine_eW'], params['gine_eb'])

    def layer_front(i, hprime, xs1):
        agg_gcn = _sc_aggregate(hprime, src, dst, None)
        agg_gine = _sc_aggregate(xs1, src, dst, e_rows)
        return _tc_b1(
            hprime, agg_gcn[0], agg_gcn[1], d0, d1, params['gcn_b%d' % i],
            xs1, agg_gine[0], agg_gine[1],
            params['gine_W1_%d' % i], params['gine_b1_%d' % i],
            params['gine_W2_%d' % i], params['gine_b2_%d' % i])

    def bn_params(i):
        return (params['bn_g_0_%d' % i], params['bn_b_0_%d' % i],
                params['bn_g_1_%d' % i], params['bn_b_1_%d' % i])

    hprime = _tc_hprime(x0, params['gcn_W0'], d0, d1)
    gcn_pre, gine_pre, st = layer_front(0, hprime, x1)
    xs0, xs1, hprime2 = _tc_b2h(gcn_pre, gine_pre, st, x0, x1, *bn_params(0),
                                params['gcn_W1'], d0, d1)
    gcn_pre, gine_pre, st = layer_front(1, hprime2, xs1)
    batch_col = batch.reshape(N, 1)
    return _tc_b2pool(gcn_pre, gine_pre, st, xs0, xs1, *bn_params(1),
                      batch_col, params['out_W'], params['out_b'])


# speedup vs baseline: 1.0240x; 1.0240x over previous
import functools

import jax
import jax.numpy as jnp
from jax import lax
from jax.experimental import pallas as pl
from jax.experimental.pallas import tpu as pltpu
from jax.experimental.pallas import tpu_sc as plsc

N = 10000
D = 128
G = 128
NC = 2
NS = 16
L = 16
BLK = 1000
CH = 80
ROWT = 624

@functools.cache
def _get_mesh():
    return plsc.VectorSubcoreMesh(core_axis_name="c", subcore_axis_name="s",
                                  num_cores=NC, num_subcores=NS)


def _worker(cid, sid):
    return cid * NS + sid


def _zero_acc(zeros_hbm, acc, sid):
    pltpu.sync_copy(zeros_hbm, acc.at[pl.ds(sid * ROWT, ROWT)])

    @pl.when(sid == NS - 1)
    def _():
        pltpu.sync_copy(zeros_hbm.at[pl.ds(0, N - NS * ROWT)],
                        acc.at[pl.ds(NS * ROWT, N - NS * ROWT)])


def _copy_out(acc, out_hbm, cid, sid):
    pltpu.sync_copy(acc.at[pl.ds(sid * ROWT, ROWT)],
                    out_hbm.at[cid, pl.ds(sid * ROWT, ROWT)])

    @pl.when(sid == NS - 1)
    def _():
        pltpu.sync_copy(acc.at[pl.ds(NS * ROWT, N - NS * ROWT)],
                        out_hbm.at[cid, pl.ds(NS * ROWT, N - NS * ROWT)])


def _deg_body(dst_hbm, zeros_hbm, out_hbm, dst0, dst1, onesv, acc,
              semd0, semd1):
    cid = lax.axis_index("c")
    sid = lax.axis_index("s")
    e_total = dst_hbm.shape[0]
    e_core = e_total // NC
    e_tile = e_core // NS
    nchunks = e_tile // CH

    ones = jnp.ones((L,), jnp.float32)
    for r in range(CH):
        onesv[r, :] = ones
    _zero_acc(zeros_hbm, acc, sid)
    plsc.subcore_barrier()

    tile_base = cid * e_core + sid * e_tile
    dstb = (dst0, dst1)
    semdb = (semd0, semd1)

    def lddst(j, b):
        pltpu.async_copy(dst_hbm.at[pl.ds(tile_base + j * CH, CH)],
                         dstb[b], semdb[b])

    def scat(j, b):
        pltpu.make_async_copy(dst_hbm.at[pl.ds(tile_base + j * CH, CH)],
                              dstb[b], semdb[b]).wait()
        pltpu.sync_copy(onesv, acc.at[dstb[b]], add=True)

    lddst(0, 0)
    lddst(1, 1)

    def step(s, carry):
        j = 2 * s
        scat(j, 0)

        @pl.when(j + 2 < nchunks)
        def _():
            lddst(j + 2, 0)

        scat(j + 1, 1)

        @pl.when(j + 3 < nchunks)
        def _():
            lddst(j + 3, 1)

        return carry

    lax.fori_loop(0, (nchunks - 1) // 2, step, 0)
    scat(nchunks - 1, 0)
    plsc.subcore_barrier()
    _copy_out(acc, out_hbm, cid, sid)


def _sc_degree(dst):
    zeros = jnp.zeros((ROWT, L), jnp.float32)
    return pl.kernel(
        _deg_body,
        out_type=jax.ShapeDtypeStruct((NC, N, L), jnp.float32),
        mesh=_get_mesh(),
        scratch_types=[
            pltpu.VMEM((CH,), jnp.int32),
            pltpu.VMEM((CH,), jnp.int32),
            pltpu.VMEM((CH, L), jnp.float32),
            pltpu.VMEM_SHARED((N, L), jnp.float32),
            pltpu.SemaphoreType.DMA,
            pltpu.SemaphoreType.DMA,
        ],
        compiler_params=pltpu.CompilerParams(use_tc_tiling_on_sc=False),
    )(dst, zeros)


def _agg_body(table_hbm, src_hbm, dst_hbm, zeros_hbm, out_hbm,
              src0, src1, dst0, dst1, dst2, dst3, rows0, rows1, acc,
              semg0, semg1, sems0, sems1,
              semd0, semd1, semd2, semd3, semsc0, semsc1):
    cid = lax.axis_index("c")
    sid = lax.axis_index("s")
    e_total = src_hbm.shape[0]
    e_core = e_total // NC
    e_tile = e_core // NS
    nchunks = e_tile // CH

    _zero_acc(zeros_hbm, acc, sid)
    tile_base = cid * e_core + sid * e_tile
    plsc.subcore_barrier()

    srcb = (src0, src1)
    dstb = (dst0, dst1, dst2, dst3)
    rowsb = (rows0, rows1)
    semgb = (semg0, semg1)
    semsb = (sems0, sems1)
    semdb = (semd0, semd1, semd2, semd3)
    semscb = (semsc0, semsc1)

    def ldsrc(j, b):
        pltpu.async_copy(src_hbm.at[pl.ds(tile_base + j * CH, CH)],
                         srcb[b], semsb[b])

    def lddst(j, b):
        pltpu.async_copy(dst_hbm.at[pl.ds(tile_base + j * CH, CH)],
                         dstb[b], semdb[b])

    def wait_scatter(rb):
        pltpu.make_async_copy(rowsb[rb], acc.at[dstb[0]], semscb[rb]).wait()

    def gather(j, rb, wait_sc):
        pltpu.make_async_copy(src_hbm.at[pl.ds(tile_base + j * CH, CH)],
                              srcb[rb], semsb[rb]).wait()
        if wait_sc is True:
            wait_scatter(rb)
        elif wait_sc is not None:
            @pl.when(wait_sc)
            def _():
                wait_scatter(rb)
        pltpu.async_copy(table_hbm.at[srcb[rb]], rowsb[rb], semgb[rb])

    def scat(j, rb, db):
        pltpu.make_async_copy(table_hbm.at[srcb[rb]],
                              rowsb[rb], semgb[rb]).wait()
        pltpu.make_async_copy(dst_hbm.at[pl.ds(tile_base + j * CH, CH)],
                              dstb[db], semdb[db]).wait()
        pltpu.async_copy(rowsb[rb], acc.at[dstb[db]], semscb[rb], add=True)

    ldsrc(0, 0)
    ldsrc(1, 1)
    lddst(0, 0)
    lddst(1, 1)
    lddst(2, 2)
    gather(0, 0, wait_sc=None)

    def step(s, carry):
        j = 4 * s
        for p in range(4):
            c = j + p
            gather(c + 1, (p + 1) % 2, wait_sc=(j > 0) if p == 0 else True)
            scat(c, p % 2, p % 4)

            @pl.when(c + 2 < nchunks)
            def _(c=c, p=p):
                ldsrc(c + 2, p % 2)

            @pl.when(c + 3 < nchunks)
            def _(c=c, p=p):
                lddst(c + 3, (p + 3) % 4)

        return carry

    lax.fori_loop(0, (nchunks - 1) // 4, step, 0)
    scat(nchunks - 1, (nchunks - 1) % 2, (nchunks - 1) % 4)
    wait_scatter(0)
    wait_scatter(1)
    plsc.subcore_barrier()
    _copy_out(acc, out_hbm, cid, sid)


def _gine_body(table_hbm, src_hbm, dst_hbm, e_hbm, zeros_hbm, out_hbm,
               src0, src1, dst0, dst1, rows0, rows1, acc,
               semg0, semg1, seme0, seme1, sems0, sems1, semd0, semd1):
    cid = lax.axis_index("c")
    sid = lax.axis_index("s")
    e_total = src_hbm.shape[0]
    e_core = e_total // NC
    e_tile = e_core // NS
    nchunks = e_tile // CH

    _zero_acc(zeros_hbm, acc, sid)
    tile_base = cid * e_core + sid * e_tile
    plsc.subcore_barrier()

    srcb = (src0, src1)
    dstb = (dst0, dst1)
    rowsb = (rows0, rows1)
    semgb = (semg0, semg1)
    semeb = (seme0, seme1)
    semsb = (sems0, sems1)
    semdb = (semd0, semd1)

    def ldidx(j, b):
        pltpu.async_copy(src_hbm.at[pl.ds(tile_base + j * CH, CH)],
                         srcb[b], semsb[b])
        pltpu.async_copy(dst_hbm.at[pl.ds(tile_base + j * CH, CH)],
                         dstb[b], semdb[b])

    def e_load(j, b):
        pltpu.async_copy(e_hbm.at[pl.ds(tile_base + j * CH, CH)],
                         rowsb[b], semeb[b])

    def ga(j, b):
        pltpu.make_async_copy(e_hbm.at[pl.ds(tile_base + j * CH, CH)],
                              rowsb[b], semeb[b]).wait()
        pltpu.make_async_copy(src_hbm.at[pl.ds(tile_base + j * CH, CH)],
                              srcb[b], semsb[b]).wait()
        pltpu.async_copy(table_hbm.at[srcb[b]], rowsb[b], semgb[b], add=True)

    def fin(j, b):
        pltpu.make_async_copy(table_hbm.at[srcb[b]],
                              rowsb[b], semgb[b]).wait()
        pltpu.make_async_copy(dst_hbm.at[pl.ds(tile_base + j * CH, CH)],
                              dstb[b], semdb[b]).wait()

        def row_relu(r, c2):
            for k in range(D // L):
                sl = pl.ds(k * L, L)
                rowsb[b][r, sl] = jnp.maximum(rowsb[b][r, sl], 0.0)
            return c2

        lax.fori_loop(0, CH, row_relu, 0, unroll=2)
        pltpu.sync_copy(rowsb[b], acc.at[dstb[b]], add=True)

    ldidx(0, 0)
    ldidx(1, 1)
    e_load(0, 0)
    e_load(1, 1)
    ga(0, 0)

    def step(s, carry):
        j = 2 * s
        ga(j + 1, 1)
        fin(j, 0)

        @pl.when(j + 2 < nchunks)
        def _():
            ldidx(j + 2, 0)
            e_load(j + 2, 0)

        fin(j + 1, 1)

        @pl.when(j + 2 < nchunks)
        def _():
            ga(j + 2, 0)

        @pl.when(j + 3 < nchunks)
        def _():
            ldidx(j + 3, 1)
            e_load(j + 3, 1)

        return carry

    lax.fori_loop(0, (nchunks - 1) // 2, step, 0)
    fin(nchunks - 1, 0)
    plsc.subcore_barrier()
    _copy_out(acc, out_hbm, cid, sid)


def _sc_aggregate(table, src, dst, e_rows):
    with_e = e_rows is not None
    zeros = jnp.zeros((ROWT, D), jnp.float32)
    e_tile = src.shape[0] // (NC * NS)
    nchunks = e_tile // CH
    assert nchunks % 2 == 1 and e_tile % CH == 0
    idx = lambda: pltpu.VMEM((CH,), jnp.int32)
    rows = lambda: pltpu.VMEM((CH, D), jnp.float32)
    sem = pltpu.SemaphoreType.DMA
    if with_e:
        scratch = ([idx(), idx(), idx(), idx(), rows(), rows(),
                    pltpu.VMEM_SHARED((N, D), jnp.float32)] + [sem] * 8)
        return pl.kernel(
            _gine_body,
            out_type=jax.ShapeDtypeStruct((NC, N, D), jnp.float32),
            mesh=_get_mesh(),
            scratch_types=scratch,
        )(table, src, dst, e_rows, zeros)
    assert (nchunks - 1) % 4 == 0
    scratch = ([idx(), idx(), idx(), idx(), idx(), idx(), rows(), rows(),
                pltpu.VMEM_SHARED((N, D), jnp.float32)] + [sem] * 10)
    return pl.kernel(
        _agg_body,
        out_type=jax.ShapeDtypeStruct((NC, N, D), jnp.float32),
        mesh=_get_mesh(),
        scratch_types=scratch,
    )(table, src, dst, zeros)


def _row_spec(cols=D):
    return pl.BlockSpec((BLK, cols), lambda i: (i, 0))


def _full_spec(shape):
    return pl.BlockSpec(shape, lambda i: tuple(0 for _ in shape))


def _dinv_of(d0_ref, d1_ref):
    deg = d0_ref[:, 0:1] + d1_ref[:, 0:1] + 1.0
    return lax.rsqrt(jnp.maximum(deg, 1.0))


def _embed_body(ea_ref, w_ref, b_ref, out_ref):
    out_ref[...] = jnp.dot(ea_ref[...], w_ref[...],
                           preferred_element_type=jnp.float32) + b_ref[...]


def _tc_embed(edge_attr, w, b):
    e, de = edge_attr.shape
    eblk = 4000
    return pl.pallas_call(
        _embed_body,
        grid=(e // eblk,),
        in_specs=[pl.BlockSpec((eblk, de), lambda i: (i, 0)),
                  _full_spec((de, D)),
                  _full_spec((1, D))],
        out_specs=pl.BlockSpec((eblk, D), lambda i: (i, 0)),
        out_shape=jax.ShapeDtypeStruct((e, D), jnp.float32),
    )(edge_attr, w, b.reshape(1, D))


def _hprime_body(x_ref, w_ref, d0_ref, d1_ref, out_ref):
    out_ref[...] = jnp.dot(x_ref[...], w_ref[...],
                           preferred_element_type=jnp.float32) * _dinv_of(d0_ref, d1_ref)


def _tc_hprime(x, w, d0, d1):
    return pl.pallas_call(
        _hprime_body,
        grid=(N // BLK,),
        in_specs=[_row_spec(), _full_spec((D, D)), _row_spec(L), _row_spec(L)],
        out_specs=_row_spec(),
        out_shape=jax.ShapeDtypeStruct((N, D), jnp.float32),
    )(x, w, d0, d1)


def _b1_body(hp_ref, a0_ref, a1_ref, d0_ref, d1_ref, bg_ref,
             x1_ref, g0_ref, g1_ref, w1_ref, b1_ref, w2_ref, b2_ref,
             gcn_ref, gine_ref, st_ref):
    i = pl.program_id(0)
    gcn = (_dinv_of(d0_ref, d1_ref) * (a0_ref[...] + a1_ref[...] + hp_ref[...])
           + bg_ref[...])
    h = x1_ref[...] + g0_ref[...] + g1_ref[...]
    h = jnp.maximum(jnp.dot(h, w1_ref[...],
                            preferred_element_type=jnp.float32) + b1_ref[...], 0.0)
    gine = jnp.dot(h, w2_ref[...],
                   preferred_element_type=jnp.float32) + b2_ref[...]
    gcn_ref[...] = gcn
    gine_ref[...] = gine
    z = jnp.zeros((1, D), jnp.float32)
    st = jnp.concatenate([
        jnp.sum(gcn, axis=0, keepdims=True),
        jnp.sum(gcn * gcn, axis=0, keepdims=True),
        jnp.sum(gine, axis=0, keepdims=True),
        jnp.sum(gine * gine, axis=0, keepdims=True),
        z, z, z, z], axis=0)

    @pl.when(i == 0)
    def _():
        st_ref[...] = jnp.zeros_like(st_ref)

    st_ref[...] += st


def _tc_b1(hprime, a0, a1, d0, d1, bg, x1, g0, g1, w1, b1, w2, b2):
    return pl.pallas_call(
        _b1_body,
        grid=(N // BLK,),
        in_specs=[_row_spec(), _row_spec(), _row_spec(),
                  _row_spec(L), _row_spec(L),
                  _full_spec((1, D)), _row_spec(), _row_spec(), _row_spec(),
                  _full_spec((D, D)), _full_spec((1, D)),
                  _full_spec((D, D)), _full_spec((1, D))],
        out_specs=[_row_spec(), _row_spec(), _full_spec((8, D))],
        out_shape=[jax.ShapeDtypeStruct((N, D), jnp.float32),
                   jax.ShapeDtypeStruct((N, D), jnp.float32),
                   jax.ShapeDtypeStruct((8, D), jnp.float32)],
    )(hprime, a0, a1, d0, d1, bg.reshape(1, D), x1, g0, g1,
      w1, b1.reshape(1, D), w2, b2.reshape(1, D))


def _pair_attention(x):
    half = x.shape[0] // 2
    xr = x.reshape(half, 2, D)
    a = xr[:, 0, :]
    b = xr[:, 1, :]
    saa = jnp.sum(a * a, axis=-1, keepdims=True)
    sab = jnp.sum(a * b, axis=-1, keepdims=True)
    sbb = jnp.sum(b * b, axis=-1, keepdims=True)
    ma = jnp.maximum(saa, sab)
    ea0 = jnp.exp(saa - ma)
    ea1 = jnp.exp(sab - ma)
    a2 = (ea0 * a + ea1 * b) / (ea0 + ea1)
    mb = jnp.maximum(sab, sbb)
    eb0 = jnp.exp(sab - mb)
    eb1 = jnp.exp(sbb - mb)
    b2 = (eb0 * a + eb1 * b) / (eb0 + eb1)
    return jnp.stack([a2, b2], axis=1).reshape(x.shape[0], D)


def _bn_attn(gcn_ref, gine_ref, st_ref, x0_ref, x1_ref,
             bg0_ref, bb0_ref, bg1_ref, bb1_ref):
    st = st_ref[...]
    n = jnp.float32(N)
    m0 = st[0:1] / n
    v0 = st[1:2] / n - m0 * m0
    m1 = st[2:3] / n
    v1 = st[3:4] / n - m1 * m1
    h0 = (gcn_ref[...] - m0) * lax.rsqrt(v0 + 1e-5) * bg0_ref[...] + bb0_ref[...]
    x0n = x0_ref[...] + jnp.maximum(h0, 0.0)
    h1 = (gine_ref[...] - m1) * lax.rsqrt(v1 + 1e-5) * bg1_ref[...] + bb1_ref[...]
    x1n = x1_ref[...] + jnp.maximum(h1, 0.0)
    return _pair_attention(x0n), _pair_attention(x1n)


def _b2h_body(gcn_ref, gine_ref, st_ref, x0_ref, x1_ref,
              bg0_ref, bb0_ref, bg1_ref, bb1_ref, w_ref, d0_ref, d1_ref,
              o0_ref, o1_ref, hp_ref):
    o0, o1 = _bn_attn(gcn_ref, gine_ref, st_ref, x0_ref, x1_ref,
                      bg0_ref, bb0_ref, bg1_ref, bb1_ref)
    o0_ref[...] = o0
    o1_ref[...] = o1
    hp_ref[...] = jnp.dot(o0, w_ref[...],
                          preferred_element_type=jnp.float32) * _dinv_of(d0_ref, d1_ref)


def _tc_b2h(gcn_pre, gine_pre, st, x0, x1, bg0, bb0, bg1, bb1, w, d0, d1):
    return pl.pallas_call(
        _b2h_body,
        grid=(N // BLK,),
        in_specs=[_row_spec(), _row_spec(), _full_spec((8, D)),
                  _row_spec(), _row_spec(),
                  _full_spec((1, D)), _full_spec((1, D)),
                  _full_spec((1, D)), _full_spec((1, D)),
                  _full_spec((D, D)), _row_spec(L), _row_spec(L)],
        out_specs=[_row_spec(), _row_spec(), _row_spec()],
        out_shape=[jax.ShapeDtypeStruct((N, D), jnp.float32),
                   jax.ShapeDtypeStruct((N, D), jnp.float32),
                   jax.ShapeDtypeStruct((N, D), jnp.float32)],
    )(gcn_pre, gine_pre, st, x0, x1,
      bg0.reshape(1, D), bb0.reshape(1, D), bg1.reshape(1, D), bb1.reshape(1, D),
      w, d0, d1)


def _b2pool_body(gcn_ref, gine_ref, st_ref, x0_ref, x1_ref,
                 bg0_ref, bb0_ref, bg1_ref, bb1_ref, b_ref, ow_ref, ob_ref,
                 out_ref, sums_ref, cnt_ref):
    i = pl.program_id(0)
    nb = pl.num_programs(0)
    o0, o1 = _bn_attn(gcn_ref, gine_ref, st_ref, x0_ref, x1_ref,
                      bg0_ref, bb0_ref, bg1_ref, bb1_ref)
    x = o0 + o1
    seg = b_ref[...]
    iota = lax.broadcasted_iota(jnp.int32, (1, G), 1)
    onehot = (seg == iota).astype(jnp.float32)
    psum = lax.dot_general(onehot, x, (((0,), (0,)), ((), ())),
                           preferred_element_type=jnp.float32)
    ones = jnp.ones((x.shape[0], 1), jnp.float32)
    pcnt = lax.dot_general(onehot, ones, (((0,), (0,)), ((), ())),
                           preferred_element_type=jnp.float32)

    @pl.when(i == 0)
    def _():
        sums_ref[...] = jnp.zeros_like(sums_ref)
        cnt_ref[...] = jnp.zeros_like(cnt_ref)

    sums_ref[...] += psum
    cnt_ref[...] += pcnt

    @pl.when(i == nb - 1)
    def _():
        pooled = sums_ref[...] / jnp.maximum(cnt_ref[...], 1.0)
        out_ref[...] = jnp.dot(pooled, ow_ref[...],
                               preferred_element_type=jnp.float32) + ob_ref[...]


def _tc_b2pool(gcn_pre, gine_pre, st, x0, x1, bg0, bb0, bg1, bb1,
               batch_col, out_w, out_b):
    out_dim = out_w.shape[1]
    out, _, _ = pl.pallas_call(
        _b2pool_body,
        grid=(N // BLK,),
        in_specs=[_row_spec(), _row_spec(), _full_spec((8, D)),
                  _row_spec(), _row_spec(),
                  _full_spec((1, D)), _full_spec((1, D)),
                  _full_spec((1, D)), _full_spec((1, D)),
                  _row_spec(1),
                  _full_spec((D, out_dim)), _full_spec((1, out_dim))],
        out_specs=[_full_spec((G, out_dim)), _full_spec((G, D)),
                   _full_spec((G, 1))],
        out_shape=[jax.ShapeDtypeStruct((G, out_dim), jnp.float32),
                   jax.ShapeDtypeStruct((G, D), jnp.float32),
                   jax.ShapeDtypeStruct((G, 1), jnp.float32)],
    )(gcn_pre, gine_pre, st, x0, x1,
      bg0.reshape(1, D), bb0.reshape(1, D), bg1.reshape(1, D), bb1.reshape(1, D),
      batch_col, out_w, out_b.reshape(1, out_dim))
    return out


def kernel(x0, x1, edge_attr, params, edge_index, batch):
    src = edge_index[0]
    dst = edge_index[1]

    deg_partials = _sc_degree(dst)
    d0, d1 = deg_partials[0], deg_partials[1]

    e_rows = _tc_embed(edge_attr, params['gine_eW'], params['gine_eb'])

    def layer_front(i, hprime, xs1):
        agg_gcn = _sc_aggregate(hprime, src, dst, None)
        agg_gine = _sc_aggregate(xs1, src, dst, e_rows)
        return _tc_b1(
            hprime, agg_gcn[0], agg_gcn[1], d0, d1, params['gcn_b%d' % i],
            xs1, agg_gine[0], agg_gine[1],
            params['gine_W1_%d' % i], params['gine_b1_%d' % i],
            params['gine_W2_%d' % i], params['gine_b2_%d' % i])

    def bn_params(i):
        return (params['bn_g_0_%d' % i], params['bn_b_0_%d' % i],
                params['bn_g_1_%d' % i], params['bn_b_1_%d' % i])

    hprime = _tc_hprime(x0, params['gcn_W0'], d0, d1)
    gcn_pre, gine_pre, st = layer_front(0, hprime, x1)
    xs0, xs1, hprime2 = _tc_b2h(gcn_pre, gine_pre, st, x0, x1, *bn_params(0),
                                params['gcn_W1'], d0, d1)
    gcn_pre, gine_pre, st = layer_front(1, hprime2, xs1)
    batch_col = batch.reshape(N, 1)
    return _tc_b2pool(gcn_pre, gine_pre, st, xs0, xs1, *bn_params(1),
                      batch_col, params['out_W'], params['out_b'])


# speedup vs baseline: 1.0242x; 1.0002x over previous
import functools

import jax
import jax.numpy as jnp
from jax import lax
from jax.experimental import pallas as pl
from jax.experimental.pallas import tpu as pltpu
from jax.experimental.pallas import tpu_sc as plsc

N = 10000
D = 128
G = 128
NC = 2
NS = 16
L = 16
BLK = 1000
CH = 80
ROWT = 624

@functools.cache
def _get_mesh():
    return plsc.VectorSubcoreMesh(core_axis_name="c", subcore_axis_name="s",
                                  num_cores=NC, num_subcores=NS)


def _worker(cid, sid):
    return cid * NS + sid


def _zero_acc(zeros_hbm, acc, sid):
    pltpu.sync_copy(zeros_hbm, acc.at[pl.ds(sid * ROWT, ROWT)])

    @pl.when(sid == NS - 1)
    def _():
        pltpu.sync_copy(zeros_hbm.at[pl.ds(0, N - NS * ROWT)],
                        acc.at[pl.ds(NS * ROWT, N - NS * ROWT)])


def _copy_out(acc, out_hbm, cid, sid):
    pltpu.sync_copy(acc.at[pl.ds(sid * ROWT, ROWT)],
                    out_hbm.at[cid, pl.ds(sid * ROWT, ROWT)])

    @pl.when(sid == NS - 1)
    def _():
        pltpu.sync_copy(acc.at[pl.ds(NS * ROWT, N - NS * ROWT)],
                        out_hbm.at[cid, pl.ds(NS * ROWT, N - NS * ROWT)])


def _deg_body(dst_hbm, zeros_hbm, out_hbm, dst0, dst1, onesv, acc,
              semd0, semd1):
    cid = lax.axis_index("c")
    sid = lax.axis_index("s")
    e_total = dst_hbm.shape[0]
    e_core = e_total // NC
    e_tile = e_core // NS
    nchunks = e_tile // CH

    ones = jnp.ones((L,), jnp.float32)
    for r in range(CH):
        onesv[r, :] = ones
    _zero_acc(zeros_hbm, acc, sid)
    plsc.subcore_barrier()

    tile_base = cid * e_core + sid * e_tile
    dstb = (dst0, dst1)
    semdb = (semd0, semd1)

    def lddst(j, b):
        pltpu.async_copy(dst_hbm.at[pl.ds(tile_base + j * CH, CH)],
                         dstb[b], semdb[b])

    def scat(j, b):
        pltpu.make_async_copy(dst_hbm.at[pl.ds(tile_base + j * CH, CH)],
                              dstb[b], semdb[b]).wait()
        pltpu.sync_copy(onesv, acc.at[dstb[b]], add=True)

    lddst(0, 0)
    lddst(1, 1)

    def step(s, carry):
        j = 2 * s
        scat(j, 0)

        @pl.when(j + 2 < nchunks)
        def _():
            lddst(j + 2, 0)

        scat(j + 1, 1)

        @pl.when(j + 3 < nchunks)
        def _():
            lddst(j + 3, 1)

        return carry

    lax.fori_loop(0, (nchunks - 1) // 2, step, 0)
    scat(nchunks - 1, 0)
    plsc.subcore_barrier()
    _copy_out(acc, out_hbm, cid, sid)


def _sc_degree(dst):
    zeros = jnp.zeros((ROWT, L), jnp.float32)
    return pl.kernel(
        _deg_body,
        out_type=jax.ShapeDtypeStruct((NC, N, L), jnp.float32),
        mesh=_get_mesh(),
        scratch_types=[
            pltpu.VMEM((CH,), jnp.int32),
            pltpu.VMEM((CH,), jnp.int32),
            pltpu.VMEM((CH, L), jnp.float32),
            pltpu.VMEM_SHARED((N, L), jnp.float32),
            pltpu.SemaphoreType.DMA,
            pltpu.SemaphoreType.DMA,
        ],
        compiler_params=pltpu.CompilerParams(use_tc_tiling_on_sc=False),
    )(dst, zeros)


def _agg_body(table_hbm, src_hbm, dst_hbm, zeros_hbm, out_hbm,
              src0, src1, dst0, dst1, dst2, dst3, rows0, rows1, acc,
              semg0, semg1, sems0, sems1,
              semd0, semd1, semd2, semd3, semsc0, semsc1):
    cid = lax.axis_index("c")
    sid = lax.axis_index("s")
    e_total = src_hbm.shape[0]
    e_core = e_total // NC
    e_tile = e_core // NS
    nchunks = e_tile // CH

    _zero_acc(zeros_hbm, acc, sid)
    tile_base = cid * e_core + sid * e_tile
    plsc.subcore_barrier()

    srcb = (src0, src1)
    dstb = (dst0, dst1, dst2, dst3)
    rowsb = (rows0, rows1)
    semgb = (semg0, semg1)
    semsb = (sems0, sems1)
    semdb = (semd0, semd1, semd2, semd3)
    semscb = (semsc0, semsc1)

    def ldsrc(j, b):
        pltpu.async_copy(src_hbm.at[pl.ds(tile_base + j * CH, CH)],
                         srcb[b], semsb[b])

    def lddst(j, b):
        pltpu.async_copy(dst_hbm.at[pl.ds(tile_base + j * CH, CH)],
                         dstb[b], semdb[b])

    def wait_scatter(rb):
        pltpu.make_async_copy(rowsb[rb], acc.at[dstb[0]], semscb[rb]).wait()

    def gather(j, rb, wait_sc):
        pltpu.make_async_copy(src_hbm.at[pl.ds(tile_base + j * CH, CH)],
                              srcb[rb], semsb[rb]).wait()
        if wait_sc is True:
            wait_scatter(rb)
        elif wait_sc is not None:
            @pl.when(wait_sc)
            def _():
                wait_scatter(rb)
        pltpu.async_copy(table_hbm.at[srcb[rb]], rowsb[rb], semgb[rb])

    def scat(j, rb, db):
        pltpu.make_async_copy(table_hbm.at[srcb[rb]],
                              rowsb[rb], semgb[rb]).wait()
        pltpu.make_async_copy(dst_hbm.at[pl.ds(tile_base + j * CH, CH)],
                              dstb[db], semdb[db]).wait()
        pltpu.async_copy(rowsb[rb], acc.at[dstb[db]], semscb[rb], add=True)

    ldsrc(0, 0)
    ldsrc(1, 1)
    lddst(0, 0)
    lddst(1, 1)
    lddst(2, 2)
    gather(0, 0, wait_sc=None)

    def step(s, carry):
        j = 4 * s
        for p in range(4):
            c = j + p
            gather(c + 1, (p + 1) % 2, wait_sc=(j > 0) if p == 0 else True)
            scat(c, p % 2, p % 4)

            @pl.when(c + 2 < nchunks)
            def _(c=c, p=p):
                ldsrc(c + 2, p % 2)

            @pl.when(c + 3 < nchunks)
            def _(c=c, p=p):
                lddst(c + 3, (p + 3) % 4)

        return carry

    lax.fori_loop(0, (nchunks - 1) // 4, step, 0)
    scat(nchunks - 1, (nchunks - 1) % 2, (nchunks - 1) % 4)
    wait_scatter(0)
    wait_scatter(1)
    plsc.subcore_barrier()
    _copy_out(acc, out_hbm, cid, sid)


def _gine_body(table_hbm, src_hbm, dst_hbm, e_hbm, zeros_hbm, out_hbm,
               src0, src1, dst0, dst1, rows0, rows1, acc,
               semg0, semg1, seme0, seme1, sems0, sems1, semd0, semd1):
    cid = lax.axis_index("c")
    sid = lax.axis_index("s")
    e_total = src_hbm.shape[0]
    e_core = e_total // NC
    e_tile = e_core // NS
    nchunks = e_tile // CH

    _zero_acc(zeros_hbm, acc, sid)
    tile_base = cid * e_core + sid * e_tile
    plsc.subcore_barrier()

    srcb = (src0, src1)
    dstb = (dst0, dst1)
    rowsb = (rows0, rows1)
    semgb = (semg0, semg1)
    semeb = (seme0, seme1)
    semsb = (sems0, sems1)
    semdb = (semd0, semd1)

    def ldidx(j, b):
        pltpu.async_copy(src_hbm.at[pl.ds(tile_base + j * CH, CH)],
                         srcb[b], semsb[b])
        pltpu.async_copy(dst_hbm.at[pl.ds(tile_base + j * CH, CH)],
                         dstb[b], semdb[b])

    def e_load(j, b):
        pltpu.async_copy(e_hbm.at[pl.ds(tile_base + j * CH, CH)],
                         rowsb[b], semeb[b])

    def ga(j, b):
        pltpu.make_async_copy(e_hbm.at[pl.ds(tile_base + j * CH, CH)],
                              rowsb[b], semeb[b]).wait()
        pltpu.make_async_copy(src_hbm.at[pl.ds(tile_base + j * CH, CH)],
                              srcb[b], semsb[b]).wait()
        pltpu.async_copy(table_hbm.at[srcb[b]], rowsb[b], semgb[b], add=True)

    def fin(j, b):
        pltpu.make_async_copy(table_hbm.at[srcb[b]],
                              rowsb[b], semgb[b]).wait()
        pltpu.make_async_copy(dst_hbm.at[pl.ds(tile_base + j * CH, CH)],
                              dstb[b], semdb[b]).wait()

        def row_relu(r, c2):
            for k in range(D // L):
                sl = pl.ds(k * L, L)
                rowsb[b][r, sl] = jnp.maximum(rowsb[b][r, sl], 0.0)
            return c2

        lax.fori_loop(0, CH, row_relu, 0, unroll=4)
        pltpu.sync_copy(rowsb[b], acc.at[dstb[b]], add=True)

    ldidx(0, 0)
    ldidx(1, 1)
    e_load(0, 0)
    e_load(1, 1)
    ga(0, 0)

    def step(s, carry):
        j = 2 * s
        ga(j + 1, 1)
        fin(j, 0)

        @pl.when(j + 2 < nchunks)
        def _():
            ldidx(j + 2, 0)
            e_load(j + 2, 0)

        fin(j + 1, 1)

        @pl.when(j + 2 < nchunks)
        def _():
            ga(j + 2, 0)

        @pl.when(j + 3 < nchunks)
        def _():
            ldidx(j + 3, 1)
            e_load(j + 3, 1)

        return carry

    lax.fori_loop(0, (nchunks - 1) // 2, step, 0)
    fin(nchunks - 1, 0)
    plsc.subcore_barrier()
    _copy_out(acc, out_hbm, cid, sid)


def _sc_aggregate(table, src, dst, e_rows):
    with_e = e_rows is not None
    zeros = jnp.zeros((ROWT, D), jnp.float32)
    e_tile = src.shape[0] // (NC * NS)
    nchunks = e_tile // CH
    assert nchunks % 2 == 1 and e_tile % CH == 0
    idx = lambda: pltpu.VMEM((CH,), jnp.int32)
    rows = lambda: pltpu.VMEM((CH, D), jnp.float32)
    sem = pltpu.SemaphoreType.DMA
    if with_e:
        scratch = ([idx(), idx(), idx(), idx(), rows(), rows(),
                    pltpu.VMEM_SHARED((N, D), jnp.float32)] + [sem] * 8)
        return pl.kernel(
            _gine_body,
            out_type=jax.ShapeDtypeStruct((NC, N, D), jnp.float32),
            mesh=_get_mesh(),
            scratch_types=scratch,
        )(table, src, dst, e_rows, zeros)
    assert (nchunks - 1) % 4 == 0
    scratch = ([idx(), idx(), idx(), idx(), idx(), idx(), rows(), rows(),
                pltpu.VMEM_SHARED((N, D), jnp.float32)] + [sem] * 10)
    return pl.kernel(
        _agg_body,
        out_type=jax.ShapeDtypeStruct((NC, N, D), jnp.float32),
        mesh=_get_mesh(),
        scratch_types=scratch,
    )(table, src, dst, zeros)


def _row_spec(cols=D):
    return pl.BlockSpec((BLK, cols), lambda i: (i, 0))


def _full_spec(shape):
    return pl.BlockSpec(shape, lambda i: tuple(0 for _ in shape))


def _dinv_of(d0_ref, d1_ref):
    deg = d0_ref[:, 0:1] + d1_ref[:, 0:1] + 1.0
    return lax.rsqrt(jnp.maximum(deg, 1.0))


def _embed_body(ea_ref, w_ref, b_ref, out_ref):
    out_ref[...] = jnp.dot(ea_ref[...], w_ref[...],
                           preferred_element_type=jnp.float32) + b_ref[...]


def _tc_embed(edge_attr, w, b):
    e, de = edge_attr.shape
    eblk = 4000
    return pl.pallas_call(
        _embed_body,
        grid=(e // eblk,),
        in_specs=[pl.BlockSpec((eblk, de), lambda i: (i, 0)),
                  _full_spec((de, D)),
                  _full_spec((1, D))],
        out_specs=pl.BlockSpec((eblk, D), lambda i: (i, 0)),
        out_shape=jax.ShapeDtypeStruct((e, D), jnp.float32),
    )(edge_attr, w, b.reshape(1, D))


def _hprime_body(x_ref, w_ref, d0_ref, d1_ref, out_ref):
    out_ref[...] = jnp.dot(x_ref[...], w_ref[...],
                           preferred_element_type=jnp.float32) * _dinv_of(d0_ref, d1_ref)


def _tc_hprime(x, w, d0, d1):
    return pl.pallas_call(
        _hprime_body,
        grid=(N // BLK,),
        in_specs=[_row_spec(), _full_spec((D, D)), _row_spec(L), _row_spec(L)],
        out_specs=_row_spec(),
        out_shape=jax.ShapeDtypeStruct((N, D), jnp.float32),
    )(x, w, d0, d1)


def _b1_body(hp_ref, a0_ref, a1_ref, d0_ref, d1_ref, bg_ref,
             x1_ref, g0_ref, g1_ref, w1_ref, b1_ref, w2_ref, b2_ref,
             gcn_ref, gine_ref, st_ref):
    i = pl.program_id(0)
    gcn = (_dinv_of(d0_ref, d1_ref) * (a0_ref[...] + a1_ref[...] + hp_ref[...])
           + bg_ref[...])
    h = x1_ref[...] + g0_ref[...] + g1_ref[...]
    h = jnp.maximum(jnp.dot(h, w1_ref[...],
                            preferred_element_type=jnp.float32) + b1_ref[...], 0.0)
    gine = jnp.dot(h, w2_ref[...],
                   preferred_element_type=jnp.float32) + b2_ref[...]
    gcn_ref[...] = gcn
    gine_ref[...] = gine
    z = jnp.zeros((1, D), jnp.float32)
    st = jnp.concatenate([
        jnp.sum(gcn, axis=0, keepdims=True),
        jnp.sum(gcn * gcn, axis=0, keepdims=True),
        jnp.sum(gine, axis=0, keepdims=True),
        jnp.sum(gine * gine, axis=0, keepdims=True),
        z, z, z, z], axis=0)

    @pl.when(i == 0)
    def _():
        st_ref[...] = jnp.zeros_like(st_ref)

    st_ref[...] += st


def _tc_b1(hprime, a0, a1, d0, d1, bg, x1, g0, g1, w1, b1, w2, b2):
    return pl.pallas_call(
        _b1_body,
        grid=(N // BLK,),
        in_specs=[_row_spec(), _row_spec(), _row_spec(),
                  _row_spec(L), _row_spec(L),
                  _full_spec((1, D)), _row_spec(), _row_spec(), _row_spec(),
                  _full_spec((D, D)), _full_spec((1, D)),
                  _full_spec((D, D)), _full_spec((1, D))],
        out_specs=[_row_spec(), _row_spec(), _full_spec((8, D))],
        out_shape=[jax.ShapeDtypeStruct((N, D), jnp.float32),
                   jax.ShapeDtypeStruct((N, D), jnp.float32),
                   jax.ShapeDtypeStruct((8, D), jnp.float32)],
    )(hprime, a0, a1, d0, d1, bg.reshape(1, D), x1, g0, g1,
      w1, b1.reshape(1, D), w2, b2.reshape(1, D))


def _pair_attention(x):
    half = x.shape[0] // 2
    xr = x.reshape(half, 2, D)
    a = xr[:, 0, :]
    b = xr[:, 1, :]
    saa = jnp.sum(a * a, axis=-1, keepdims=True)
    sab = jnp.sum(a * b, axis=-1, keepdims=True)
    sbb = jnp.sum(b * b, axis=-1, keepdims=True)
    ma = jnp.maximum(saa, sab)
    ea0 = jnp.exp(saa - ma)
    ea1 = jnp.exp(sab - ma)
    a2 = (ea0 * a + ea1 * b) / (ea0 + ea1)
    mb = jnp.maximum(sab, sbb)
    eb0 = jnp.exp(sab - mb)
    eb1 = jnp.exp(sbb - mb)
    b2 = (eb0 * a + eb1 * b) / (eb0 + eb1)
    return jnp.stack([a2, b2], axis=1).reshape(x.shape[0], D)


def _bn_attn(gcn_ref, gine_ref, st_ref, x0_ref, x1_ref,
             bg0_ref, bb0_ref, bg1_ref, bb1_ref):
    st = st_ref[...]
    n = jnp.float32(N)
    m0 = st[0:1] / n
    v0 = st[1:2] / n - m0 * m0
    m1 = st[2:3] / n
    v1 = st[3:4] / n - m1 * m1
    h0 = (gcn_ref[...] - m0) * lax.rsqrt(v0 + 1e-5) * bg0_ref[...] + bb0_ref[...]
    x0n = x0_ref[...] + jnp.maximum(h0, 0.0)
    h1 = (gine_ref[...] - m1) * lax.rsqrt(v1 + 1e-5) * bg1_ref[...] + bb1_ref[...]
    x1n = x1_ref[...] + jnp.maximum(h1, 0.0)
    return _pair_attention(x0n), _pair_attention(x1n)


def _b2h_body(gcn_ref, gine_ref, st_ref, x0_ref, x1_ref,
              bg0_ref, bb0_ref, bg1_ref, bb1_ref, w_ref, d0_ref, d1_ref,
              o0_ref, o1_ref, hp_ref):
    o0, o1 = _bn_attn(gcn_ref, gine_ref, st_ref, x0_ref, x1_ref,
                      bg0_ref, bb0_ref, bg1_ref, bb1_ref)
    o0_ref[...] = o0
    o1_ref[...] = o1
    hp_ref[...] = jnp.dot(o0, w_ref[...],
                          preferred_element_type=jnp.float32) * _dinv_of(d0_ref, d1_ref)


def _tc_b2h(gcn_pre, gine_pre, st, x0, x1, bg0, bb0, bg1, bb1, w, d0, d1):
    return pl.pallas_call(
        _b2h_body,
        grid=(N // BLK,),
        in_specs=[_row_spec(), _row_spec(), _full_spec((8, D)),
                  _row_spec(), _row_spec(),
                  _full_spec((1, D)), _full_spec((1, D)),
                  _full_spec((1, D)), _full_spec((1, D)),
                  _full_spec((D, D)), _row_spec(L), _row_spec(L)],
        out_specs=[_row_spec(), _row_spec(), _row_spec()],
        out_shape=[jax.ShapeDtypeStruct((N, D), jnp.float32),
                   jax.ShapeDtypeStruct((N, D), jnp.float32),
                   jax.ShapeDtypeStruct((N, D), jnp.float32)],
    )(gcn_pre, gine_pre, st, x0, x1,
      bg0.reshape(1, D), bb0.reshape(1, D), bg1.reshape(1, D), bb1.reshape(1, D),
      w, d0, d1)


def _b2pool_body(gcn_ref, gine_ref, st_ref, x0_ref, x1_ref,
                 bg0_ref, bb0_ref, bg1_ref, bb1_ref, b_ref, ow_ref, ob_ref,
                 out_ref, sums_ref, cnt_ref):
    i = pl.program_id(0)
    nb = pl.num_programs(0)
    o0, o1 = _bn_attn(gcn_ref, gine_ref, st_ref, x0_ref, x1_ref,
                      bg0_ref, bb0_ref, bg1_ref, bb1_ref)
    x = o0 + o1
    seg = b_ref[...]
    iota = lax.broadcasted_iota(jnp.int32, (1, G), 1)
    onehot = (seg == iota).astype(jnp.float32)
    psum = lax.dot_general(onehot, x, (((0,), (0,)), ((), ())),
                           preferred_element_type=jnp.float32)
    ones = jnp.ones((x.shape[0], 1), jnp.float32)
    pcnt = lax.dot_general(onehot, ones, (((0,), (0,)), ((), ())),
                           preferred_element_type=jnp.float32)

    @pl.when(i == 0)
    def _():
        sums_ref[...] = jnp.zeros_like(sums_ref)
        cnt_ref[...] = jnp.zeros_like(cnt_ref)

    sums_ref[...] += psum
    cnt_ref[...] += pcnt

    @pl.when(i == nb - 1)
    def _():
        pooled = sums_ref[...] / jnp.maximum(cnt_ref[...], 1.0)
        out_ref[...] = jnp.dot(pooled, ow_ref[...],
                               preferred_element_type=jnp.float32) + ob_ref[...]


def _tc_b2pool(gcn_pre, gine_pre, st, x0, x1, bg0, bb0, bg1, bb1,
               batch_col, out_w, out_b):
    out_dim = out_w.shape[1]
    out, _, _ = pl.pallas_call(
        _b2pool_body,
        grid=(N // BLK,),
        in_specs=[_row_spec(), _row_spec(), _full_spec((8, D)),
                  _row_spec(), _row_spec(),
                  _full_spec((1, D)), _full_spec((1, D)),
                  _full_spec((1, D)), _full_spec((1, D)),
                  _row_spec(1),
                  _full_spec((D, out_dim)), _full_spec((1, out_dim))],
        out_specs=[_full_spec((G, out_dim)), _full_spec((G, D)),
                   _full_spec((G, 1))],
        out_shape=[jax.ShapeDtypeStruct((G, out_dim), jnp.float32),
                   jax.ShapeDtypeStruct((G, D), jnp.float32),
                   jax.ShapeDtypeStruct((G, 1), jnp.float32)],
    )(gcn_pre, gine_pre, st, x0, x1,
      bg0.reshape(1, D), bb0.reshape(1, D), bg1.reshape(1, D), bb1.reshape(1, D),
      batch_col, out_w, out_b.reshape(1, out_dim))
    return out


def kernel(x0, x1, edge_attr, params, edge_index, batch):
    src = edge_index[0]
    dst = edge_index[1]

    deg_partials = _sc_degree(dst)
    d0, d1 = deg_partials[0], deg_partials[1]

    e_rows = _tc_embed(edge_attr, params['gine_eW'], params['gine_eb'])

    def layer_front(i, hprime, xs1):
        agg_gcn = _sc_aggregate(hprime, src, dst, None)
        agg_gine = _sc_aggregate(xs1, src, dst, e_rows)
        return _tc_b1(
            hprime, agg_gcn[0], agg_gcn[1], d0, d1, params['gcn_b%d' % i],
            xs1, agg_gine[0], agg_gine[1],
            params['gine_W1_%d' % i], params['gine_b1_%d' % i],
            params['gine_W2_%d' % i], params['gine_b2_%d' % i])

    def bn_params(i):
        return (params['bn_g_0_%d' % i], params['bn_b_0_%d' % i],
                params['bn_g_1_%d' % i], params['bn_b_1_%d' % i])

    hprime = _tc_hprime(x0, params['gcn_W0'], d0, d1)
    gcn_pre, gine_pre, st = layer_front(0, hprime, x1)
    xs0, xs1, hprime2 = _tc_b2h(gcn_pre, gine_pre, st, x0, x1, *bn_params(0),
                                params['gcn_W1'], d0, d1)
    gcn_pre, gine_pre, st = layer_front(1, hprime2, xs1)
    batch_col = batch.reshape(N, 1)
    return _tc_b2pool(gcn_pre, gine_pre, st, xs0, xs1, *bn_params(1),
                      batch_col, params['out_W'], params['out_b'])
